# baseline (speedup 1.0000x reference)
import jax
import jax.numpy as jnp
from jax.experimental import pallas as pl
from jax.experimental.pallas import tpu as pltpu

_B = 1024
_K = 1024
_D = 64
_BB = 32
_NBLK = _B // _BB
_NBUF = 4


def _som_kernel(x_ref, lmt_ref, qd_ref, out_hbm, scratch, sems):
    i = pl.program_id(0)
    s = jax.lax.rem(i, _NBUF)

    @pl.when(i >= _NBUF)
    def _wait_reuse():
        old = (i - _NBUF) * _BB
        pltpu.make_async_copy(
            scratch.at[s], out_hbm.at[pl.ds(old, _BB)], sems.at[s]
        ).wait()

    x = x_ref[...]
    lmt = lmt_ref[...]
    xlm = jax.lax.dot_general(
        x, lmt, (((1,), (0,)), ((), ())), preferred_element_type=jnp.float32
    )
    x2 = jnp.sum(x * x, axis=1, keepdims=True)
    lm2 = jnp.sum(lmt * lmt, axis=0, keepdims=True)
    dist = x2 + lm2 - 2.0 * xlm
    dmin = jnp.min(dist, axis=1, keepdims=True)
    iota = jax.lax.broadcasted_iota(jnp.int32, dist.shape, 1)
    idx = jnp.min(jnp.where(dist == dmin, iota, _K), axis=1, keepdims=True)
    onehot = (iota == idx).astype(jnp.float32)
    h = jax.lax.dot_general(
        onehot, qd_ref[...], (((1,), (0,)), ((), ())),
        preferred_element_type=jnp.float32,
    )
    for b in range(_BB):
        scratch[s, b, :, :] = h[b, None, :] * (x[b, :, None] - lmt)
    pltpu.make_async_copy(
        scratch.at[s], out_hbm.at[pl.ds(i * _BB, _BB)], sems.at[s]
    ).start()

    @pl.when(i == _NBLK - 1)
    def _drain():
        for j in range(_NBUF):
            step = _NBLK - _NBUF + j
            pltpu.make_async_copy(
                scratch.at[j], out_hbm.at[pl.ds(step * _BB, _BB)], sems.at[j]
            ).wait()


@jax.jit
def kernel(x, landmarks, qd):
    out_t = pl.pallas_call(
        _som_kernel,
        grid=(_NBLK,),
        in_specs=[
            pl.BlockSpec((_BB, _D), lambda i: (i, 0)),
            pl.BlockSpec((_D, _K), lambda i: (0, 0)),
            pl.BlockSpec((_K, _K), lambda i: (0, 0)),
        ],
        out_specs=pl.BlockSpec(memory_space=pl.ANY),
        out_shape=jax.ShapeDtypeStruct((_B, _D, _K), jnp.float32),
        scratch_shapes=[
            pltpu.VMEM((_NBUF, _BB, _D, _K), jnp.float32),
            pltpu.SemaphoreType.DMA((_NBUF,)),
        ],
    )(x, landmarks.T, qd)
    return jnp.swapaxes(out_t, 1, 2)


# speedup vs baseline: 1.0203x; 1.0203x over previous
import jax
import jax.numpy as jnp
from jax.experimental import pallas as pl
from jax.experimental.pallas import tpu as pltpu

_B = 1024
_K = 1024
_D = 64
_BB = 32
_NBLK = _B // _BB
_NBUF = 4


_HALF = _BB // 2


def _som_kernel(x_ref, lmt_ref, qd_ref, out_hbm, scratch, sems):
    i = pl.program_id(0)
    s = jax.lax.rem(i, _NBUF)

    @pl.when(i >= _NBUF)
    def _wait_reuse():
        old = (i - _NBUF) * _BB
        for half in range(2):
            pltpu.make_async_copy(
                scratch.at[s, pl.ds(half * _HALF, _HALF)],
                out_hbm.at[pl.ds(old + half * _HALF, _HALF)],
                sems.at[s, half],
            ).wait()

    x = x_ref[...]
    lmt = lmt_ref[...]
    xlm = jax.lax.dot_general(
        x, lmt, (((1,), (0,)), ((), ())), preferred_element_type=jnp.float32
    )
    x2 = jnp.sum(x * x, axis=1, keepdims=True)
    lm2 = jnp.sum(lmt * lmt, axis=0, keepdims=True)
    dist = x2 + lm2 - 2.0 * xlm
    dmin = jnp.min(dist, axis=1, keepdims=True)
    iota = jax.lax.broadcasted_iota(jnp.int32, dist.shape, 1)
    idx = jnp.min(jnp.where(dist == dmin, iota, _K), axis=1, keepdims=True)
    onehot = (iota == idx).astype(jnp.float32)
    h = jax.lax.dot_general(
        onehot, qd_ref[...], (((1,), (0,)), ((), ())),
        preferred_element_type=jnp.float32,
    )
    for half in range(2):
        lo = half * _HALF
        scratch[s, lo:lo + _HALF] = h[lo:lo + _HALF, None, :] * (
            x[lo:lo + _HALF, :, None] - lmt[None, :, :]
        )
        pltpu.make_async_copy(
            scratch.at[s, pl.ds(lo, _HALF)],
            out_hbm.at[pl.ds(i * _BB + lo, _HALF)],
            sems.at[s, half],
        ).start()

    @pl.when(i == _NBLK - 1)
    def _drain():
        for j in range(_NBUF):
            step = _NBLK - _NBUF + j
            for half in range(2):
                pltpu.make_async_copy(
                    scratch.at[j, pl.ds(half * _HALF, _HALF)],
                    out_hbm.at[pl.ds(step * _BB + half * _HALF, _HALF)],
                    sems.at[j, half],
                ).wait()


@jax.jit
def kernel(x, landmarks, qd):
    out_t = pl.pallas_call(
        _som_kernel,
        grid=(_NBLK,),
        in_specs=[
            pl.BlockSpec((_BB, _D), lambda i: (i, 0)),
            pl.BlockSpec((_D, _K), lambda i: (0, 0)),
            pl.BlockSpec((_K, _K), lambda i: (0, 0)),
        ],
        out_specs=pl.BlockSpec(memory_space=pl.ANY),
        out_shape=jax.ShapeDtypeStruct((_B, _D, _K), jnp.float32),
        scratch_shapes=[
            pltpu.VMEM((_NBUF, _BB, _D, _K), jnp.float32),
            pltpu.SemaphoreType.DMA((_NBUF, 2)),
        ],
    )(x, landmarks.T, qd)
    return jnp.swapaxes(out_t, 1, 2)
